# baseline (speedup 1.0000x reference)
import jax
import jax.numpy as jnp
from jax import lax
from jax.experimental import pallas as pl
from jax.experimental.pallas import tpu as pltpu, tpu_sc as plsc

NC = 2
NS = 16
NW = NC * NS

DIM = 64
ROW = 128
CHUNK = 800


def _gather_body(idx_hbm, table_hbm, out_hbm, idx_v0, idx_v1, rows_v0,
                 rows_v1, si, sg, so):
    wid = lax.axis_index("s") * NC + lax.axis_index("c")
    tot = idx_hbm.shape[0]
    per_w = tot // NW
    n = per_w // CHUNK
    base = wid * per_w
    idx_v = [idx_v0, idx_v1]
    rows_v = [rows_v0, rows_v1]

    def idx_start(j, b):
        pltpu.async_copy(
            idx_hbm.at[pl.ds(base + j * CHUNK, CHUNK)], idx_v[b], si.at[b])

    def idx_wait(b):
        pltpu.make_async_copy(
            idx_hbm.at[pl.ds(0, CHUNK)], idx_v[b], si.at[b]).wait()

    def gather_start(b):
        pltpu.async_copy(table_hbm.at[idx_v[b]], rows_v[b], sg.at[b])

    def gather_wait(b):
        pltpu.make_async_copy(
            table_hbm.at[idx_v[b]], rows_v[b], sg.at[b]).wait()

    def out_start(j, b):
        pltpu.async_copy(
            rows_v[b],
            out_hbm.at[pl.ds(base + j * CHUNK, CHUNK), pl.ds(0, DIM)],
            so.at[b])

    def out_wait(b):
        pltpu.make_async_copy(
            rows_v[b], out_hbm.at[pl.ds(0, CHUNK), pl.ds(0, DIM)],
            so.at[b]).wait()

    idx_start(0, 0)
    idx_start(1, 1)
    idx_wait(0)
    gather_start(0)

    @pl.loop(0, n, step=2)
    def _(j0):
        for t in range(2):
            j = j0 + t
            b = t
            nb = 1 - t
            gather_wait(b)
            out_start(j, b)
            @pl.when(j + 2 < n)
            def _():
                idx_start(j + 2, b)
            @pl.when(j + 1 < n)
            def _():
                idx_wait(nb)
                @pl.when(j >= 1)
                def _():
                    out_wait(nb)
                gather_start(nb)

    out_wait(0)
    out_wait(1)


def _embedding_gather(flat_idx, table):
    tot = flat_idx.shape[0]
    mesh = plsc.VectorSubcoreMesh(core_axis_name="c", subcore_axis_name="s")
    run = pl.kernel(
        _gather_body,
        out_type=jax.ShapeDtypeStruct((tot, ROW), jnp.float32),
        mesh=mesh,
        scratch_types=[
            pltpu.VMEM((CHUNK,), jnp.int32),
            pltpu.VMEM((CHUNK,), jnp.int32),
            pltpu.VMEM((CHUNK, DIM), jnp.float32),
            pltpu.VMEM((CHUNK, DIM), jnp.float32),
            pltpu.SemaphoreType.DMA((2,)),
            pltpu.SemaphoreType.DMA((2,)),
            pltpu.SemaphoreType.DMA((2,)),
        ],
        compiler_params=pltpu.CompilerParams(use_tc_tiling_on_sc=False),
    )
    return run(flat_idx, table)


def kernel(x, table):
    b, l = x.shape
    outp = _embedding_gather(x.reshape(-1), table)
    return outp[:, :DIM].reshape(b, l, DIM)
